# Initial kernel scaffold; baseline (speedup 1.0000x reference)
import jax
import jax.numpy as jnp
from jax.experimental import pallas as pl


def kernel(features, labels):
    raise NotImplementedError("write your pallas kernel here")


# speedup vs baseline: 4.2103x; 4.2103x over previous
import jax
import jax.numpy as jnp
from jax.experimental import pallas as pl
from jax.experimental.pallas import tpu as pltpu

_N = 16384
_D = 256
_C = 128
_BR = 2048
_NB = _N // _BR
_INTRA_W = 1.0
_INTER_W = 0.8
_PREC = jax.lax.Precision.HIGHEST


def _dot(a, b, dims):
    return jax.lax.dot_general(a, b, (dims, ((), ())),
                               preferred_element_type=jnp.float32,
                               precision=_PREC)


def _loss_body(x_ref, lab_ref, out_ref, sums, counts, cent, minsq, acc):
    p = pl.program_id(0)
    i = pl.program_id(1)

    lab = lab_ref[...]
    classes = jax.lax.broadcasted_iota(jnp.int32, (_BR, _C), 1)
    onehot = (lab == classes).astype(jnp.float32)

    @pl.when((p == 0) & (i == 0))
    def _init():
        sums[...] = jnp.zeros_like(sums)
        counts[...] = jnp.zeros_like(counts)

    @pl.when(p == 0)
    def _accumulate_sums():
        x = x_ref[...]
        sums[...] += _dot(onehot, x, ((0,), (0,)))
        counts[...] += _dot(onehot, jnp.ones((_BR, 1), jnp.float32),
                            ((0,), (0,)))

    @pl.when((p == 1) & (i == 0))
    def _centers_and_inter():
        cen = sums[...] / jnp.maximum(counts[...], 1.0)
        cent[...] = cen
        csq = cen * cen
        ones_row = jnp.ones((1, _D), jnp.float32)
        cn_col = _dot(csq, ones_row, ((1,), (1,)))
        cn_row = _dot(ones_row, csq, ((1,), (1,)))
        gram = _dot(cen, cen, ((1,), (1,)))
        sq = cn_col + cn_row - 2.0 * gram
        ii = jax.lax.broadcasted_iota(jnp.int32, (_C, _C), 0)
        jj = jax.lax.broadcasted_iota(jnp.int32, (_C, _C), 1)
        off = ii != jj
        minsq[0, 0] = jnp.min(jnp.where(off, sq, jnp.inf))
        acc[0, 0] = 0.0

    @pl.when(p == 1)
    def _intra():
        x = x_ref[...]
        cgath = _dot(onehot, cent[...], ((1,), (0,)))
        diff = x - cgath
        d2 = jnp.sum(diff * diff, axis=1)
        acc[0, 0] += jnp.sum(jnp.sqrt(d2))

    @pl.when((p == 1) & (i == _NB - 1))
    def _finish():
        intra_loss = acc[0, 0] / _N
        inter_loss = -0.5 * jnp.log(minsq[0, 0])
        loss = _INTRA_W * intra_loss + _INTER_W * inter_loss
        out_ref[...] = loss.reshape(1, 1)


def kernel(features, labels):
    labels2d = labels.reshape(_N, 1)
    out = pl.pallas_call(
        _loss_body,
        grid=(2, _NB),
        in_specs=[
            pl.BlockSpec((_BR, _D), lambda p, i: (i, 0)),
            pl.BlockSpec((_BR, 1), lambda p, i: (i, 0)),
        ],
        out_specs=pl.BlockSpec((1, 1), lambda p, i: (0, 0)),
        out_shape=jax.ShapeDtypeStruct((1, 1), jnp.float32),
        scratch_shapes=[
            pltpu.VMEM((_C, _D), jnp.float32),
            pltpu.VMEM((_C, 1), jnp.float32),
            pltpu.VMEM((_C, _D), jnp.float32),
            pltpu.SMEM((1, 1), jnp.float32),
            pltpu.SMEM((1, 1), jnp.float32),
        ],
    )(features, labels2d)
    return out[0, 0]


# speedup vs baseline: 6.2473x; 1.4838x over previous
import jax
import jax.numpy as jnp
from jax.experimental import pallas as pl
from jax.experimental.pallas import tpu as pltpu

_N = 16384
_D = 256
_C = 128
_BR = 2048
_NB = _N // _BR
_INTRA_W = 1.0
_INTER_W = 0.8
_PREC = jax.lax.Precision.DEFAULT


def _dot(a, b, dims):
    return jax.lax.dot_general(a, b, (dims, ((), ())),
                               preferred_element_type=jnp.float32,
                               precision=_PREC)


def _loss_body(x_ref, lab_ref, out_ref, sums, counts, cent, minsq, acc):
    p = pl.program_id(0)
    i = pl.program_id(1)

    lab = lab_ref[...]
    classes = jax.lax.broadcasted_iota(jnp.int32, (_BR, _C), 1)
    onehot = (lab == classes).astype(jnp.float32)

    @pl.when((p == 0) & (i == 0))
    def _init():
        sums[...] = jnp.zeros_like(sums)
        counts[...] = jnp.zeros_like(counts)

    @pl.when(p == 0)
    def _accumulate_sums():
        x = x_ref[...]
        sums[...] += _dot(onehot, x, ((0,), (0,)))
        counts[...] += _dot(onehot, jnp.ones((_BR, 1), jnp.float32),
                            ((0,), (0,)))

    @pl.when((p == 1) & (i == 0))
    def _centers_and_inter():
        cen = sums[...] / jnp.maximum(counts[...], 1.0)
        cent[...] = cen
        csq = cen * cen
        ones_row = jnp.ones((1, _D), jnp.float32)
        cn_col = _dot(csq, ones_row, ((1,), (1,)))
        cn_row = _dot(ones_row, csq, ((1,), (1,)))
        gram = _dot(cen, cen, ((1,), (1,)))
        sq = cn_col + cn_row - 2.0 * gram
        ii = jax.lax.broadcasted_iota(jnp.int32, (_C, _C), 0)
        jj = jax.lax.broadcasted_iota(jnp.int32, (_C, _C), 1)
        off = ii != jj
        minsq[0, 0] = jnp.min(jnp.where(off, sq, jnp.inf))
        acc[0, 0] = 0.0

    @pl.when(p == 1)
    def _intra():
        x = x_ref[...]
        cgath = _dot(onehot, cent[...], ((1,), (0,)))
        diff = x - cgath
        d2 = jnp.sum(diff * diff, axis=1)
        acc[0, 0] += jnp.sum(jnp.sqrt(d2))

    @pl.when((p == 1) & (i == _NB - 1))
    def _finish():
        intra_loss = acc[0, 0] / _N
        inter_loss = -0.5 * jnp.log(minsq[0, 0])
        loss = _INTRA_W * intra_loss + _INTER_W * inter_loss
        out_ref[...] = loss.reshape(1, 1)


def kernel(features, labels):
    labels2d = labels.reshape(_N, 1)
    out = pl.pallas_call(
        _loss_body,
        grid=(2, _NB),
        in_specs=[
            pl.BlockSpec((_BR, _D), lambda p, i: (i, 0)),
            pl.BlockSpec((_BR, 1), lambda p, i: (i, 0)),
        ],
        out_specs=pl.BlockSpec((1, 1), lambda p, i: (0, 0)),
        out_shape=jax.ShapeDtypeStruct((1, 1), jnp.float32),
        scratch_shapes=[
            pltpu.VMEM((_C, _D), jnp.float32),
            pltpu.VMEM((_C, 1), jnp.float32),
            pltpu.VMEM((_C, _D), jnp.float32),
            pltpu.SMEM((1, 1), jnp.float32),
            pltpu.SMEM((1, 1), jnp.float32),
        ],
    )(features, labels2d)
    return out[0, 0]


# speedup vs baseline: 6.6164x; 1.0591x over previous
import jax
import jax.numpy as jnp
from jax.experimental import pallas as pl
from jax.experimental.pallas import tpu as pltpu

_N = 16384
_D = 256
_C = 128
_BR = 2048
_NB = _N // _BR
_INTRA_W = 1.0
_INTER_W = 0.8
_PREC = jax.lax.Precision.DEFAULT


def _dot(a, b, dims):
    return jax.lax.dot_general(a, b, (dims, ((), ())),
                               preferred_element_type=jnp.float32,
                               precision=_PREC)


def _loss_body(x_ref, lab_ref, out_ref, sums, counts, cent, minsq, acc, xstash):
    p = pl.program_id(0)
    i = pl.program_id(1)

    lab = lab_ref[...]
    classes = jax.lax.broadcasted_iota(jnp.int32, (_BR, _C), 1)
    onehot = (lab == classes).astype(jnp.float32)

    @pl.when((p == 0) & (i == 0))
    def _init():
        sums[...] = jnp.zeros_like(sums)
        counts[...] = jnp.zeros_like(counts)

    @pl.when(p == 0)
    def _accumulate_sums():
        x = x_ref[...]
        xstash[pl.ds(i * _BR, _BR), :] = x
        sums[...] += _dot(onehot, x, ((0,), (0,)))
        counts[...] += _dot(onehot, jnp.ones((_BR, 1), jnp.float32),
                            ((0,), (0,)))

    @pl.when((p == 1) & (i == 0))
    def _centers_and_inter():
        cen = sums[...] / jnp.maximum(counts[...], 1.0)
        cent[...] = cen
        csq = cen * cen
        ones_row = jnp.ones((1, _D), jnp.float32)
        cn_col = _dot(csq, ones_row, ((1,), (1,)))
        cn_row = _dot(ones_row, csq, ((1,), (1,)))
        gram = _dot(cen, cen, ((1,), (1,)))
        sq = cn_col + cn_row - 2.0 * gram
        ii = jax.lax.broadcasted_iota(jnp.int32, (_C, _C), 0)
        jj = jax.lax.broadcasted_iota(jnp.int32, (_C, _C), 1)
        off = ii != jj
        minsq[0, 0] = jnp.min(jnp.where(off, sq, jnp.inf))
        acc[0, 0] = 0.0

    @pl.when(p == 1)
    def _intra():
        x = xstash[pl.ds(i * _BR, _BR), :]
        cgath = _dot(onehot, cent[...], ((1,), (0,)))
        diff = x - cgath
        d2 = jnp.sum(diff * diff, axis=1)
        acc[0, 0] += jnp.sum(jnp.sqrt(d2))

    @pl.when((p == 1) & (i == _NB - 1))
    def _finish():
        intra_loss = acc[0, 0] / _N
        inter_loss = -0.5 * jnp.log(minsq[0, 0])
        loss = _INTRA_W * intra_loss + _INTER_W * inter_loss
        out_ref[...] = loss.reshape(1, 1)


def kernel(features, labels):
    labels2d = labels.reshape(_N, 1)
    out = pl.pallas_call(
        _loss_body,
        grid=(2, _NB),
        in_specs=[
            pl.BlockSpec((_BR, _D), lambda p, i: ((1 - p) * i, 0)),
            pl.BlockSpec((_BR, 1), lambda p, i: (i, 0)),
        ],
        out_specs=pl.BlockSpec((1, 1), lambda p, i: (0, 0)),
        out_shape=jax.ShapeDtypeStruct((1, 1), jnp.float32),
        scratch_shapes=[
            pltpu.VMEM((_C, _D), jnp.float32),
            pltpu.VMEM((_C, 1), jnp.float32),
            pltpu.VMEM((_C, _D), jnp.float32),
            pltpu.SMEM((1, 1), jnp.float32),
            pltpu.SMEM((1, 1), jnp.float32),
            pltpu.VMEM((_N, _D), jnp.float32),
        ],
    )(features, labels2d)
    return out[0, 0]


# speedup vs baseline: 7.5030x; 1.1340x over previous
import jax
import jax.numpy as jnp
from jax.experimental import pallas as pl
from jax.experimental.pallas import tpu as pltpu

_N = 16384
_D = 256
_C = 128
_BR = 4096
_NB = _N // _BR
_INTRA_W = 1.0
_INTER_W = 0.8
_PREC = jax.lax.Precision.DEFAULT


def _dot(a, b, dims):
    return jax.lax.dot_general(a, b, (dims, ((), ())),
                               preferred_element_type=jnp.float32,
                               precision=_PREC)


def _loss_body(x_ref, lab_ref, out_ref, sums, counts, cent, minsq, acc, xstash):
    p = pl.program_id(0)
    i = pl.program_id(1)

    lab = lab_ref[...]
    classes = jax.lax.broadcasted_iota(jnp.int32, (_BR, _C), 1)
    onehot = (lab == classes).astype(jnp.float32)

    @pl.when((p == 0) & (i == 0))
    def _init():
        sums[...] = jnp.zeros_like(sums)
        counts[...] = jnp.zeros_like(counts)

    @pl.when(p == 0)
    def _accumulate_sums():
        x = x_ref[...]
        xstash[pl.ds(i * _BR, _BR), :] = x
        sums[...] += _dot(onehot, x, ((0,), (0,)))
        counts[...] += _dot(onehot, jnp.ones((_BR, 1), jnp.float32),
                            ((0,), (0,)))

    @pl.when((p == 1) & (i == 0))
    def _centers_and_inter():
        cen = sums[...] / jnp.maximum(counts[...], 1.0)
        cent[...] = cen
        csq = cen * cen
        ones_row = jnp.ones((1, _D), jnp.float32)
        cn_col = _dot(csq, ones_row, ((1,), (1,)))
        cn_row = _dot(ones_row, csq, ((1,), (1,)))
        gram = _dot(cen, cen, ((1,), (1,)))
        sq = cn_col + cn_row - 2.0 * gram
        ii = jax.lax.broadcasted_iota(jnp.int32, (_C, _C), 0)
        jj = jax.lax.broadcasted_iota(jnp.int32, (_C, _C), 1)
        off = ii != jj
        minsq[0, 0] = jnp.min(jnp.where(off, sq, jnp.inf))
        acc[0, 0] = 0.0

    @pl.when(p == 1)
    def _intra():
        x = xstash[pl.ds(i * _BR, _BR), :]
        cgath = _dot(onehot, cent[...], ((1,), (0,)))
        diff = x - cgath
        d2 = jnp.sum(diff * diff, axis=1)
        acc[0, 0] += jnp.sum(jnp.sqrt(d2))

    @pl.when((p == 1) & (i == _NB - 1))
    def _finish():
        intra_loss = acc[0, 0] / _N
        inter_loss = -0.5 * jnp.log(minsq[0, 0])
        loss = _INTRA_W * intra_loss + _INTER_W * inter_loss
        out_ref[...] = loss.reshape(1, 1)


def kernel(features, labels):
    labels2d = labels.reshape(_N, 1)
    out = pl.pallas_call(
        _loss_body,
        grid=(2, _NB),
        in_specs=[
            pl.BlockSpec((_BR, _D), lambda p, i: ((1 - p) * i, 0)),
            pl.BlockSpec((_BR, 1), lambda p, i: (i, 0)),
        ],
        out_specs=pl.BlockSpec((1, 1), lambda p, i: (0, 0)),
        out_shape=jax.ShapeDtypeStruct((1, 1), jnp.float32),
        scratch_shapes=[
            pltpu.VMEM((_C, _D), jnp.float32),
            pltpu.VMEM((_C, 1), jnp.float32),
            pltpu.VMEM((_C, _D), jnp.float32),
            pltpu.SMEM((1, 1), jnp.float32),
            pltpu.SMEM((1, 1), jnp.float32),
            pltpu.VMEM((_N, _D), jnp.float32),
        ],
    )(features, labels2d)
    return out[0, 0]


# speedup vs baseline: 7.6060x; 1.0137x over previous
import jax
import jax.numpy as jnp
from jax.experimental import pallas as pl
from jax.experimental.pallas import tpu as pltpu

_N = 16384
_D = 256
_C = 128
_BR = 8192
_NB = _N // _BR
_INTRA_W = 1.0
_INTER_W = 0.8
_PREC = jax.lax.Precision.DEFAULT


def _dot(a, b, dims):
    return jax.lax.dot_general(a, b, (dims, ((), ())),
                               preferred_element_type=jnp.float32,
                               precision=_PREC)


def _loss_body(x_ref, lab_ref, out_ref, sums, counts, cent, minsq, acc, xstash):
    p = pl.program_id(0)
    i = pl.program_id(1)

    lab = lab_ref[...]
    classes = jax.lax.broadcasted_iota(jnp.int32, (_BR, _C), 1)
    onehot = (lab == classes).astype(jnp.float32)

    @pl.when((p == 0) & (i == 0))
    def _init():
        sums[...] = jnp.zeros_like(sums)
        counts[...] = jnp.zeros_like(counts)

    @pl.when(p == 0)
    def _accumulate_sums():
        x = x_ref[...]
        xstash[pl.ds(i * _BR, _BR), :] = x
        sums[...] += _dot(onehot, x, ((0,), (0,)))
        counts[...] += _dot(onehot, jnp.ones((_BR, 1), jnp.float32),
                            ((0,), (0,)))

    @pl.when((p == 1) & (i == 0))
    def _centers_and_inter():
        cen = sums[...] / jnp.maximum(counts[...], 1.0)
        cent[...] = cen
        csq = cen * cen
        ones_row = jnp.ones((1, _D), jnp.float32)
        cn_col = _dot(csq, ones_row, ((1,), (1,)))
        cn_row = _dot(ones_row, csq, ((1,), (1,)))
        gram = _dot(cen, cen, ((1,), (1,)))
        sq = cn_col + cn_row - 2.0 * gram
        ii = jax.lax.broadcasted_iota(jnp.int32, (_C, _C), 0)
        jj = jax.lax.broadcasted_iota(jnp.int32, (_C, _C), 1)
        off = ii != jj
        minsq[0, 0] = jnp.min(jnp.where(off, sq, jnp.inf))
        acc[0, 0] = 0.0

    @pl.when(p == 1)
    def _intra():
        x = xstash[pl.ds(i * _BR, _BR), :]
        cgath = _dot(onehot, cent[...], ((1,), (0,)))
        diff = x - cgath
        d2 = jnp.sum(diff * diff, axis=1)
        acc[0, 0] += jnp.sum(jnp.sqrt(d2))

    @pl.when((p == 1) & (i == _NB - 1))
    def _finish():
        intra_loss = acc[0, 0] / _N
        inter_loss = -0.5 * jnp.log(minsq[0, 0])
        loss = _INTRA_W * intra_loss + _INTER_W * inter_loss
        out_ref[...] = loss.reshape(1, 1)


def kernel(features, labels):
    labels2d = labels.reshape(_N, 1)
    out = pl.pallas_call(
        _loss_body,
        grid=(2, _NB),
        in_specs=[
            pl.BlockSpec((_BR, _D), lambda p, i: ((1 - p) * i, 0)),
            pl.BlockSpec((_BR, 1), lambda p, i: (i, 0)),
        ],
        out_specs=pl.BlockSpec((1, 1), lambda p, i: (0, 0)),
        out_shape=jax.ShapeDtypeStruct((1, 1), jnp.float32),
        scratch_shapes=[
            pltpu.VMEM((_C, _D), jnp.float32),
            pltpu.VMEM((_C, 1), jnp.float32),
            pltpu.VMEM((_C, _D), jnp.float32),
            pltpu.SMEM((1, 1), jnp.float32),
            pltpu.SMEM((1, 1), jnp.float32),
            pltpu.VMEM((_N, _D), jnp.float32),
        ],
    )(features, labels2d)
    return out[0, 0]


# speedup vs baseline: 12.2757x; 1.6139x over previous
import jax
import jax.numpy as jnp
from jax.experimental import pallas as pl
from jax.experimental.pallas import tpu as pltpu

_N = 16384
_D = 256
_C = 128
_BR = 8192
_NB = _N // _BR
_INTRA_W = 1.0
_INTER_W = 0.8
_PREC = jax.lax.Precision.DEFAULT


def _dot(a, b, dims):
    return jax.lax.dot_general(a, b, (dims, ((), ())),
                               preferred_element_type=jnp.float32,
                               precision=_PREC)


def _loss_body(x_ref, lab_ref, out_ref, sums, counts, cent, minsq, acc, xstash):
    p = pl.program_id(0)
    i = pl.program_id(1)

    lab = lab_ref[...].reshape(1, _BR)
    classes = jax.lax.broadcasted_iota(jnp.int32, (_C, _BR), 0)
    onehot_t = (lab == classes).astype(jnp.float32)

    @pl.when((p == 0) & (i == 0))
    def _init():
        sums[...] = jnp.zeros_like(sums)
        counts[...] = jnp.zeros_like(counts)

    @pl.when(p == 0)
    def _accumulate_sums():
        x = x_ref[...]
        xstash[pl.ds(i * _BR, _BR), :] = x
        sums[...] += _dot(onehot_t, x, ((1,), (0,)))
        counts[...] += jnp.sum(onehot_t, axis=1, keepdims=True)

    @pl.when((p == 1) & (i == 0))
    def _centers_and_inter():
        cen = sums[...] / jnp.maximum(counts[...], 1.0)
        cent[...] = cen
        csq = cen * cen
        ones_row = jnp.ones((1, _D), jnp.float32)
        cn_col = _dot(csq, ones_row, ((1,), (1,)))
        cn_row = _dot(ones_row, csq, ((1,), (1,)))
        gram = _dot(cen, cen, ((1,), (1,)))
        sq = cn_col + cn_row - 2.0 * gram
        ii = jax.lax.broadcasted_iota(jnp.int32, (_C, _C), 0)
        jj = jax.lax.broadcasted_iota(jnp.int32, (_C, _C), 1)
        off = ii != jj
        minsq[0, 0] = jnp.min(jnp.where(off, sq, jnp.inf))
        acc[0, 0] = 0.0

    @pl.when(p == 1)
    def _intra():
        x = xstash[pl.ds(i * _BR, _BR), :]
        cgath = _dot(onehot_t, cent[...], ((0,), (0,)))
        diff = x - cgath
        d2 = jnp.sum(diff * diff, axis=1)
        acc[0, 0] += jnp.sum(jnp.sqrt(d2))

    @pl.when((p == 1) & (i == _NB - 1))
    def _finish():
        intra_loss = acc[0, 0] / _N
        inter_loss = -0.5 * jnp.log(minsq[0, 0])
        loss = _INTRA_W * intra_loss + _INTER_W * inter_loss
        out_ref[...] = loss.reshape(1, 1)


def kernel(features, labels):
    out = pl.pallas_call(
        _loss_body,
        grid=(2, _NB),
        in_specs=[
            pl.BlockSpec((_BR, _D), lambda p, i: ((1 - p) * i, 0)),
            pl.BlockSpec((_BR,), lambda p, i: (i,)),
        ],
        out_specs=pl.BlockSpec((1, 1), lambda p, i: (0, 0)),
        out_shape=jax.ShapeDtypeStruct((1, 1), jnp.float32),
        scratch_shapes=[
            pltpu.VMEM((_C, _D), jnp.float32),
            pltpu.VMEM((_C, 1), jnp.float32),
            pltpu.VMEM((_C, _D), jnp.float32),
            pltpu.SMEM((1, 1), jnp.float32),
            pltpu.SMEM((1, 1), jnp.float32),
            pltpu.VMEM((_N, _D), jnp.float32),
        ],
    )(features, labels)
    return out[0, 0]
